# baseline (speedup 1.0000x reference)
import jax
import jax.numpy as jnp
from jax import lax
from jax.experimental import pallas as pl
from jax.experimental.pallas import tpu as pltpu
from jax.experimental.pallas import tpu_sc as plsc

NUM_POS = 47360
NUM_NEG = 236800
EMB = 128

N_SC = 56832
SC_BASE = NUM_NEG - N_SC
NW = 32
CHUNK = 128
NCHUNKS = N_SC // CHUNK
CHUNK_OUT = CHUNK // 16
MAX_CHUNKS_PER_W = (NCHUNKS + NW - 1) // NW

GRID = 37
BP = NUM_POS // GRID
BNH = SC_BASE // GRID

_DN = (((1,), (1,)), ((), ()))


def _sc_body(nu_hbm, nv_hbm, scores_hbm, ub0, vb0, ub1, vb1, sb, sem0, sem1):
    wid = lax.axis_index("c") * 16 + lax.axis_index("s")
    bufs = ((ub0, vb0, sem0), (ub1, vb1, sem1))
    def start_fetch(t):
        j = wid + t * NW
        ub, vb, sem = bufs[t % 2]

        @pl.when(j < NCHUNKS)
        def _():
            base = (SC_BASE + j * CHUNK) * EMB
            pltpu.async_copy(nu_hbm.at[pl.ds(base, CHUNK * EMB)], ub, sem)
            pltpu.async_copy(nv_hbm.at[pl.ds(base, CHUNK * EMB)], vb, sem)

    start_fetch(0)
    for t in range(MAX_CHUNKS_PER_W):
        if t + 1 < MAX_CHUNKS_PER_W:
            start_fetch(t + 1)
        j = wid + t * NW
        ub, vb, sem = bufs[t % 2]

        @pl.when(j < NCHUNKS)
        def _():
            base = (SC_BASE + j * CHUNK) * EMB
            pltpu.make_async_copy(nu_hbm.at[pl.ds(base, CHUNK * EMB)], ub, sem).wait()
            pltpu.make_async_copy(nv_hbm.at[pl.ds(base, CHUNK * EMB)], vb, sem).wait()

            def row_body(r, c2):
                o = r * EMB
                s0 = (ub[pl.ds(o, 16)] * vb[pl.ds(o, 16)]
                      + ub[pl.ds(o + 16, 16)] * vb[pl.ds(o + 16, 16)])
                s1 = (ub[pl.ds(o + 32, 16)] * vb[pl.ds(o + 32, 16)]
                      + ub[pl.ds(o + 48, 16)] * vb[pl.ds(o + 48, 16)])
                s2 = (ub[pl.ds(o + 64, 16)] * vb[pl.ds(o + 64, 16)]
                      + ub[pl.ds(o + 80, 16)] * vb[pl.ds(o + 80, 16)])
                s3 = (ub[pl.ds(o + 96, 16)] * vb[pl.ds(o + 96, 16)]
                      + ub[pl.ds(o + 112, 16)] * vb[pl.ds(o + 112, 16)])
                sb[r, :] = (s0 + s1) + (s2 + s3)
                return c2

            lax.fori_loop(0, CHUNK, row_body, 0)
            pltpu.sync_copy(sb, scores_hbm.at[pl.ds(j * CHUNK, CHUNK)])


def _sc_scores(nu, nv):
    return pl.kernel(
        _sc_body,
        out_type=jax.ShapeDtypeStruct((N_SC, 16), jnp.float32),
        mesh=plsc.VectorSubcoreMesh(core_axis_name="c", subcore_axis_name="s",
                                    num_cores=2, num_subcores=16),
        scratch_types=[
            pltpu.VMEM((CHUNK * EMB,), jnp.float32),
            pltpu.VMEM((CHUNK * EMB,), jnp.float32),
            pltpu.VMEM((CHUNK * EMB,), jnp.float32),
            pltpu.VMEM((CHUNK * EMB,), jnp.float32),
            pltpu.VMEM((CHUNK, 16), jnp.float32),
            pltpu.SemaphoreType.DMA,
            pltpu.SemaphoreType.DMA,
        ],
    )(nu.reshape(-1), nv.reshape(-1))


def _tc_body(pu, pv, nu, nv, out_ref, accp_ref, accn_ref):
    i = pl.program_id(0)

    @pl.when(i == 0)
    def _():
        accp_ref[...] = jnp.zeros_like(accp_ref)
        accn_ref[...] = jnp.zeros_like(accn_ref)

    ones = jnp.ones((1, EMB), jnp.float32)

    p = pu[...] * pv[...]
    sp = lax.dot_general(ones, p, _DN, preferred_element_type=jnp.float32)
    sp = jnp.clip(sp, -6.0, 6.0)
    accp_ref[...] += jnp.log1p(jnp.exp(-sp))

    n = nu[...] * nv[...]
    sn = lax.dot_general(ones, n, _DN, preferred_element_type=jnp.float32)
    sn = jnp.clip(sn, -6.0, 6.0)
    accn_ref[...] += jnp.log1p(jnp.exp(sn))

    @pl.when(i == GRID - 1)
    def _():
        out_ref[0] = (jnp.sum(accp_ref[...]) * (1.0 / NUM_POS)
                      + jnp.sum(accn_ref[...]) * (1.0 / NUM_NEG))


def _tc_part(pu, pv, nu, nv):
    return pl.pallas_call(
        _tc_body,
        grid=(GRID,),
        in_specs=[
            pl.BlockSpec((BP, EMB), lambda i: (i, 0)),
            pl.BlockSpec((BP, EMB), lambda i: (i, 0)),
            pl.BlockSpec((BNH, EMB), lambda i: (i, 0)),
            pl.BlockSpec((BNH, EMB), lambda i: (i, 0)),
        ],
        out_specs=pl.BlockSpec(memory_space=pltpu.MemorySpace.SMEM),
        out_shape=jax.ShapeDtypeStruct((1,), jnp.float32),
        scratch_shapes=[
            pltpu.VMEM((1, BP), jnp.float32),
            pltpu.VMEM((1, BNH), jnp.float32),
        ],
    )(pu, pv, nu, nv)


def _comb_body(tcs_ref, sc_ref, out_ref):
    ones16 = jnp.ones((1, 16), jnp.float32)
    s = lax.dot_general(ones16, sc_ref[...], _DN,
                        preferred_element_type=jnp.float32)
    s = jnp.clip(s, -6.0, 6.0)
    f = jnp.log1p(jnp.exp(s))
    out_ref[0] = tcs_ref[0] + jnp.sum(f) * (1.0 / NUM_NEG)


def _combine(tc_scalar, scores):
    return pl.pallas_call(
        _comb_body,
        in_specs=[
            pl.BlockSpec(memory_space=pltpu.MemorySpace.SMEM),
            pl.BlockSpec((N_SC, 16), lambda: (0, 0)),
        ],
        out_specs=pl.BlockSpec(memory_space=pltpu.MemorySpace.SMEM),
        out_shape=jax.ShapeDtypeStruct((1,), jnp.float32),
    )(tc_scalar, scores)


def kernel(emb_pos_u, emb_pos_v, emb_neg_u, emb_neg_v):
    scores = _sc_scores(emb_neg_u, emb_neg_v)
    tc_scalar = _tc_part(emb_pos_u, emb_pos_v, emb_neg_u, emb_neg_v)
    return _combine(tc_scalar, scores)[0]


# speedup vs baseline: 1.4626x; 1.4626x over previous
import jax
import jax.numpy as jnp
from jax import lax
from jax.experimental import pallas as pl
from jax.experimental.pallas import tpu as pltpu

NUM_POS = 47360
NUM_NEG = 236800
EMB = 128
GRID = 16
BP = NUM_POS // GRID
BN = NUM_NEG // GRID

_DN = (((1,), (1,)), ((), ()))


def _body(pu, pv, nu, nv, out_ref, accp_ref, accn_ref):
    i = pl.program_id(0)

    @pl.when(i == 0)
    def _():
        accp_ref[...] = jnp.zeros_like(accp_ref)
        accn_ref[...] = jnp.zeros_like(accn_ref)

    ones = jnp.ones((1, EMB), jnp.float32)

    p = pu[...] * pv[...]
    sp = lax.dot_general(ones, p, _DN, preferred_element_type=jnp.float32)
    sp = jnp.clip(sp, -6.0, 6.0)
    accp_ref[...] += jnp.log1p(jnp.exp(-sp))

    n = nu[...] * nv[...]
    sn = lax.dot_general(ones, n, _DN, preferred_element_type=jnp.float32)
    sn = jnp.clip(sn, -6.0, 6.0)
    accn_ref[...] += jnp.log1p(jnp.exp(sn))

    @pl.when(i == GRID - 1)
    def _():
        out_ref[0] = (jnp.sum(accp_ref[...]) * (1.0 / NUM_POS)
                      + jnp.sum(accn_ref[...]) * (1.0 / NUM_NEG))


def kernel(emb_pos_u, emb_pos_v, emb_neg_u, emb_neg_v):
    loss = pl.pallas_call(
        _body,
        grid=(GRID,),
        in_specs=[
            pl.BlockSpec((BP, EMB), lambda i: (i, 0)),
            pl.BlockSpec((BP, EMB), lambda i: (i, 0)),
            pl.BlockSpec((BN, EMB), lambda i: (i, 0)),
            pl.BlockSpec((BN, EMB), lambda i: (i, 0)),
        ],
        out_specs=pl.BlockSpec(memory_space=pltpu.MemorySpace.SMEM),
        out_shape=jax.ShapeDtypeStruct((1,), jnp.float32),
        scratch_shapes=[
            pltpu.VMEM((1, BP), jnp.float32),
            pltpu.VMEM((1, BN), jnp.float32),
        ],
    )(emb_pos_u, emb_pos_v, emb_neg_u, emb_neg_v)
    return loss[0]
